# Initial kernel scaffold; baseline (speedup 1.0000x reference)
import jax
import jax.numpy as jnp
from jax.experimental import pallas as pl


def kernel(vecs_C, vecs_N, vecs_O):
    raise NotImplementedError("write your pallas kernel here")


# speedup vs baseline: 24.9670x; 24.9670x over previous
import functools
import math

import jax
import jax.numpy as jnp
from jax.experimental import pallas as pl
from jax.experimental.pallas import tpu as pltpu

N_GRID = 48
GRID = 0.5
SHIFT = N_GRID * 0.5 - 0.5
XW = 12
N_ATOMS = 1024
_E2 = math.exp(2.0)

SUB = 8
LANE = 288


def _splat_kernel(vecs_ref, out_ref, *, radii):
    ch = pl.program_id(0)

    s = jax.lax.broadcasted_iota(jnp.int32, (SUB, LANE), 0)
    c = jax.lax.broadcasted_iota(jnp.int32, (SUB, LANE), 1)
    flat = s * LANE + c
    ycoord = (flat // N_GRID).astype(jnp.float32) * GRID
    zcoord = (flat % N_GRID).astype(jnp.float32) * GRID
    xoff = jax.lax.broadcasted_iota(jnp.int32, (XW, 1, 1), 0).astype(
        jnp.float32) * GRID

    r = jnp.where(ch == 0, radii[0],
                  jnp.where(ch == 1, radii[1], radii[2])).astype(jnp.float32)
    r2 = r * r
    inv_r2 = -2.0 / r2
    qa = 4.0 / (_E2 * r2)
    qb = -12.0 / (_E2 * r)
    qc = 9.0 / _E2
    r15 = 1.5 * r

    out_ref[...] = jnp.zeros_like(out_ref)

    def body(i, _):
        vx = vecs_ref[0, 0, 3 * i] + SHIFT
        vy = vecs_ref[0, 0, 3 * i + 1] + SHIFT
        vz = vecs_ref[0, 0, 3 * i + 2] + SHIFT
        x0 = jnp.clip(jnp.floor(2.0 * vx).astype(jnp.int32) - 5, 0,
                      N_GRID - XW)
        dyz2 = (vy - ycoord) ** 2 + (vz - zcoord) ** 2
        dx = vx - (x0.astype(jnp.float32) * GRID + xoff)
        d2 = dx * dx + dyz2[None, :, :]
        d = jnp.sqrt(d2)
        f1 = jnp.exp(inv_r2 * d2)
        f2 = qa * d2 + qb * d + qc
        m = jnp.where(d < r, f1, jnp.where(d < r15, f2, 0.0))
        out_ref[0, pl.ds(x0, XW), :, :] += m
        return 0

    jax.lax.fori_loop(0, N_ATOMS, body, 0)


@jax.jit
def kernel(vecs_C, vecs_N, vecs_O):
    vecs = jnp.stack([vecs_C, vecs_N, vecs_O],
                     axis=0).reshape(3, 1, 3 * N_ATOMS)
    radii = (1.7, 1.55, 1.52)
    out = pl.pallas_call(
        functools.partial(_splat_kernel, radii=radii),
        grid=(3,),
        in_specs=[
            pl.BlockSpec((1, 1, 3 * N_ATOMS), lambda ch: (ch, 0, 0),
                         memory_space=pltpu.SMEM),
        ],
        out_specs=pl.BlockSpec((1, N_GRID, SUB, LANE),
                               lambda ch: (ch, 0, 0, 0)),
        out_shape=jax.ShapeDtypeStruct((3, N_GRID, SUB, LANE), jnp.float32),
    )(vecs)
    return out.reshape(3, N_GRID, N_GRID, N_GRID)
